# baseline (speedup 1.0000x reference)
import functools

import jax
import jax.numpy as jnp
from jax import lax
from jax.experimental import pallas as pl
from jax.experimental.pallas import tpu as pltpu
from jax.experimental.pallas import tpu_sc as plsc

N_NODES = 10000
D = 128
NC = 2
NS = 16
NW = NC * NS
CHUNK = 128
G = 3
ZROWS = 632

_HI = jax.lax.Precision.HIGHEST


def _make_msg_gather(cpw):
    mesh = plsc.VectorSubcoreMesh(core_axis_name="c", subcore_axis_name="s")
    ng = cpw // G
    assert cpw % G == 0

    @functools.partial(
        pl.kernel,
        out_type=jax.ShapeDtypeStruct((NW * cpw * CHUNK, D), jnp.float32),
        mesh=mesh,
        scratch_types=[
            pltpu.VMEM((3, G, CHUNK), jnp.int32),
            pltpu.VMEM((3, CHUNK, D), jnp.float32),
            pltpu.VMEM_SHARED((N_NODES + 8, D), jnp.float32),
            pltpu.SemaphoreType.DMA((3,)),
            pltpu.SemaphoreType.DMA((3,)),
            pltpu.SemaphoreType.DMA,
        ],
    )
    def gather_k(table, src_idx, zeros, msg, srcr, rows, tab, sem_g, sem_w,
                 sem_i):
        c = lax.axis_index("c")
        s = lax.axis_index("s")
        w = c * NS + s
        pltpu.sync_copy(src_idx.at[w].at[0], srcr.at[0])
        pltpu.async_copy(src_idx.at[w].at[1], srcr.at[1], sem_i)
        @pl.when(s < 15)
        def _stage():
            pltpu.sync_copy(table.at[pl.ds(s * 640, 640)],
                            tab.at[pl.ds(s * 640, 640)])

        @pl.when(s == 15)
        def _stage_tail():
            pltpu.sync_copy(table.at[pl.ds(9600, 400)],
                            tab.at[pl.ds(9600, 400)])
            pltpu.sync_copy(zeros.at[pl.ds(0, 8)],
                            tab.at[pl.ds(N_NODES, 8)])
        plsc.subcore_barrier()

        pltpu.async_copy(tab.at[srcr.at[0].at[0]], rows.at[0], sem_g.at[0])

        def grp(g, carry):
            gp = g % 3
            gn = (g + 1) % 3
            gq = (g + 2) % 3
            for r in range(G):
                k = g * G + r
                b = r
                bn = (r + 1) % 3
                @pl.when(k >= 2)
                def _drain():
                    pltpu.make_async_copy(table.at[pl.ds(0, CHUNK)],
                                          rows.at[bn], sem_w.at[bn]).wait()

                if r < G - 1:
                    pltpu.async_copy(tab.at[srcr.at[gp].at[r + 1]],
                                     rows.at[bn], sem_g.at[bn])
                else:
                    @pl.when(g + 1 < ng)
                    def _cross():
                        pltpu.make_async_copy(
                            src_idx.at[0].at[0], srcr.at[gn],
                            sem_i).wait()
                        pltpu.async_copy(tab.at[srcr.at[gn].at[0]],
                                         rows.at[bn], sem_g.at[bn])

                    @pl.when(g + 2 < ng)
                    def _pref():
                        pltpu.async_copy(src_idx.at[w].at[g + 2],
                                         srcr.at[gq], sem_i)
                pltpu.make_async_copy(table.at[pl.ds(0, CHUNK)],
                                      rows.at[b], sem_g.at[b]).wait()
                pltpu.async_copy(
                    rows.at[b],
                    msg.at[pl.ds((w * cpw + k) * CHUNK, CHUNK)],
                    sem_w.at[b])
            return carry

        lax.fori_loop(0, ng, grp, 0)
        for m in (cpw - 2, cpw - 1):
            pltpu.make_async_copy(table.at[pl.ds(0, CHUNK)],
                                  rows.at[m % 3], sem_w.at[m % 3]).wait()

    return gather_k


def _make_msg_scatter(cpw):
    mesh = plsc.VectorSubcoreMesh(core_axis_name="c", subcore_axis_name="s")
    ng = cpw // G
    assert cpw % G == 0

    @functools.partial(
        pl.kernel,
        out_type=jax.ShapeDtypeStruct((NC, N_NODES, D), jnp.float32),
        mesh=mesh,
        scratch_types=[
            pltpu.VMEM((3, G, CHUNK), jnp.int32),
            pltpu.VMEM((3, CHUNK, D), jnp.float32),
            pltpu.VMEM_SHARED((N_NODES, D), jnp.float32),
            pltpu.SemaphoreType.DMA((3,)),
            pltpu.SemaphoreType.DMA((3,)),
            pltpu.SemaphoreType.DMA,
        ],
    )
    def scatter_k(msg, dst_idx, zeros, out, dstr, rows, acc, sem_r, sem_s,
                  sem_i):
        c = lax.axis_index("c")
        s = lax.axis_index("s")
        w = c * NS + s
        base = w * cpw
        pltpu.sync_copy(dst_idx.at[w].at[0], dstr.at[0])
        pltpu.async_copy(dst_idx.at[w].at[1], dstr.at[1], sem_i)
        @pl.when(s < 15)
        def _zero():
            pltpu.sync_copy(zeros, acc.at[pl.ds(s * ZROWS, ZROWS)])

        @pl.when(s == 15)
        def _zero_tail():
            pltpu.sync_copy(zeros.at[pl.ds(0, 520)],
                            acc.at[pl.ds(9480, 520)])
        plsc.subcore_barrier()

        pltpu.async_copy(msg.at[pl.ds(base * CHUNK, CHUNK)], rows.at[0],
                         sem_r.at[0])

        def grp(g, carry):
            gp = g % 3
            gn = (g + 1) % 3
            gq = (g + 2) % 3
            for r in range(G):
                k = g * G + r
                b = r
                bn = (r + 1) % 3
                @pl.when(k >= 2)
                def _drain():
                    pltpu.make_async_copy(msg.at[pl.ds(0, CHUNK)],
                                          rows.at[bn], sem_s.at[bn]).wait()

                @pl.when(k + 1 < cpw)
                def _next():
                    pltpu.async_copy(
                        msg.at[pl.ds((base + k + 1) * CHUNK, CHUNK)],
                        rows.at[bn], sem_r.at[bn])

                if r == G - 1:
                    @pl.when(g + 1 < ng)
                    def _cross():
                        pltpu.make_async_copy(
                            dst_idx.at[0].at[0], dstr.at[gn],
                            sem_i).wait()

                    @pl.when(g + 2 < ng)
                    def _pref():
                        pltpu.async_copy(dst_idx.at[w].at[g + 2],
                                         dstr.at[gq], sem_i)
                pltpu.make_async_copy(msg.at[pl.ds(0, CHUNK)],
                                      rows.at[b], sem_r.at[b]).wait()
                pltpu.async_copy(rows.at[b], acc.at[dstr.at[gp].at[r]],
                                 sem_s.at[b], add=True)
            return carry

        lax.fori_loop(0, ng, grp, 0)
        for m in (cpw - 2, cpw - 1):
            pltpu.make_async_copy(msg.at[pl.ds(0, CHUNK)],
                                  rows.at[m % 3], sem_s.at[m % 3]).wait()
        plsc.subcore_barrier()
        @pl.when(s < 15)
        def _pub():
            pltpu.sync_copy(acc.at[pl.ds(s * ZROWS, ZROWS)],
                            out.at[c].at[pl.ds(s * ZROWS, ZROWS)])

        @pl.when(s == 15)
        def _pub_tail():
            pltpu.sync_copy(acc.at[pl.ds(9480, 520)],
                            out.at[c].at[pl.ds(9480, 520)])

    return scatter_k


_BLK = 2000


def _mid_body(p_ref, w1_ref, b1_ref, w2_ref, o_ref):
    a = p_ref[0] + p_ref[1]
    h = jnp.maximum(
        jax.lax.dot(a, w1_ref[...], precision=_HI) + b1_ref[...], 0.0)
    o_ref[...] = jax.lax.dot(h, w2_ref[...], precision=_HI)


def _dense_mid(p, W1, b1, W2):
    n = N_NODES
    return pl.pallas_call(
        _mid_body,
        grid=(n // _BLK,),
        in_specs=[
            pl.BlockSpec((NC, _BLK, D), lambda i: (0, i, 0)),
            pl.BlockSpec((D, D), lambda i: (0, 0)),
            pl.BlockSpec((1, D), lambda i: (0, 0)),
            pl.BlockSpec((D, D), lambda i: (0, 0)),
        ],
        out_specs=pl.BlockSpec((_BLK, D), lambda i: (i, 0)),
        out_shape=jax.ShapeDtypeStruct((n, D), jnp.float32),
    )(p, W1, b1.reshape(1, D), W2)


def _fin_body(n, p_ref, b2_ref, wp1_ref, bp1_ref, wp2_ref, bp2_ref, o_ref,
              acc_ref):
    i = pl.program_id(0)

    @pl.when(i == 0)
    def _zero():
        acc_ref[...] = jnp.zeros_like(acc_ref)

    h2 = jnp.maximum(p_ref[0] + p_ref[1] + b2_ref[...], 0.0)
    acc_ref[...] += jnp.sum(h2, axis=0, keepdims=True)

    @pl.when(i == pl.num_programs(0) - 1)
    def _head():
        g = acc_ref[...] * (1.0 / n)
        hp = jnp.maximum(
            jax.lax.dot(g, wp1_ref[...], precision=_HI) + bp1_ref[...], 0.0)
        o_ref[...] = jax.lax.dot(hp, wp2_ref[...], precision=_HI) \
            + bp2_ref[...]


def _dense_final(p, b2, Wp1, bp1, Wp2, bp2):
    n = N_NODES
    return pl.pallas_call(
        functools.partial(_fin_body, n),
        grid=(n // _BLK,),
        in_specs=[
            pl.BlockSpec((NC, _BLK, D), lambda i: (0, i, 0)),
            pl.BlockSpec((1, D), lambda i: (0, 0)),
            pl.BlockSpec((D, D), lambda i: (0, 0)),
            pl.BlockSpec((1, D), lambda i: (0, 0)),
            pl.BlockSpec((D, 16), lambda i: (0, 0)),
            pl.BlockSpec((1, 16), lambda i: (0, 0)),
        ],
        out_specs=pl.BlockSpec((1, 16), lambda i: (0, 0)),
        out_shape=jax.ShapeDtypeStruct((1, 16), jnp.float32),
        scratch_shapes=[pltpu.VMEM((1, D), jnp.float32)],
    )(p, b2.reshape(1, D), Wp1, bp1.reshape(1, D), Wp2, bp2.reshape(1, 16))


def kernel(x, edge_index, W1, b1, W2, b2, Wp1, bp1, Wp2, bp2):
    src = edge_index[0]
    dst = edge_index[1]
    e = src.shape[0]
    cpw = -(-e // (NW * CHUNK))
    cpw = -(-cpw // G) * G
    e_pad = NW * cpw * CHUNK
    pad = e_pad - e
    src_p = jnp.concatenate(
        [src, jnp.full((pad,), N_NODES, jnp.int32)]
    ).reshape(NW, cpw // G, G, CHUNK)
    dst_p = jnp.concatenate(
        [dst, jnp.zeros((pad,), jnp.int32)]).reshape(NW, cpw // G, G, CHUNK)
    zeros = jnp.zeros((ZROWS, D), jnp.float32)

    msg_gather = _make_msg_gather(cpw)
    msg_scatter = _make_msg_scatter(cpw)
    p1 = msg_scatter(msg_gather(x, src_p, zeros), dst_p, zeros)
    hw2 = _dense_mid(p1, W1, b1, W2)
    p2 = msg_scatter(msg_gather(hw2, src_p, zeros), dst_p, zeros)
    out = _dense_final(p2, b2, Wp1, bp1, Wp2, bp2)
    return out.reshape(16)


# speedup vs baseline: 1.0735x; 1.0735x over previous
import functools

import jax
import jax.numpy as jnp
from jax import lax
from jax.experimental import pallas as pl
from jax.experimental.pallas import tpu as pltpu
from jax.experimental.pallas import tpu_sc as plsc

N_NODES = 10000
D = 128
NC = 2
NS = 16
NW = NC * NS
CHUNK = 128
G = 8
ZROWS = 632
N_ACC = NS * ZROWS

_HI = jax.lax.Precision.HIGHEST


def _make_layer(cpw):
    mesh = plsc.VectorSubcoreMesh(core_axis_name="c", subcore_axis_name="s")
    ng = cpw // G
    assert cpw % G == 0

    @functools.partial(
        pl.kernel,
        out_type=(jax.ShapeDtypeStruct((NC, N_ACC, D), jnp.float32),
                  jax.ShapeDtypeStruct((NW * cpw * CHUNK, D), jnp.float32)),
        mesh=mesh,
        scratch_types=[
            pltpu.VMEM((2, G, CHUNK), jnp.int32),
            pltpu.VMEM((cpw, CHUNK), jnp.int32),
            pltpu.VMEM((2, CHUNK, D), jnp.float32),
            pltpu.VMEM_SHARED((N_ACC, D), jnp.float32),
            pltpu.SemaphoreType.DMA((2,)),
            pltpu.SemaphoreType.DMA((2,)),
            pltpu.SemaphoreType.DMA,
        ],
    )
    def layer_k(table, src_idx, dst_idx, zeros, out, msg, srcr, dst_v,
                rows, spm, sem_a, sem_b, sem_i):
        c = lax.axis_index("c")
        s = lax.axis_index("s")
        w = c * NS + s
        base = w * cpw
        pltpu.sync_copy(src_idx.at[w].at[pl.ds(0, G)], srcr.at[0])
        pltpu.async_copy(src_idx.at[w].at[pl.ds(G, G)], srcr.at[1], sem_i)
        pltpu.sync_copy(dst_idx.at[w], dst_v)
        @pl.when(s < 15)
        def _stage():
            pltpu.sync_copy(table.at[pl.ds(s * 640, 640)],
                            spm.at[pl.ds(s * 640, 640)])

        @pl.when(s == 15)
        def _stage_tail():
            pltpu.sync_copy(table.at[pl.ds(9600, 400)],
                            spm.at[pl.ds(9600, 400)])
        plsc.subcore_barrier()

        pltpu.async_copy(spm.at[srcr.at[0].at[0]], rows.at[0], sem_a.at[0])

        def grp(g, carry):
            gp = g % 2
            gn = (g + 1) % 2
            for r in range(G):
                k = g * G + r
                b = r % 2
                @pl.when(k >= 1)
                def _drain():
                    pltpu.make_async_copy(table.at[pl.ds(0, CHUNK)],
                                          rows.at[1 - b],
                                          sem_b.at[1 - b]).wait()

                if r < G - 1:
                    pltpu.async_copy(spm.at[srcr.at[gp].at[r + 1]],
                                     rows.at[1 - b], sem_a.at[1 - b])
                else:
                    @pl.when(g + 1 < ng)
                    def _cross():
                        pltpu.make_async_copy(
                            src_idx.at[0].at[pl.ds(0, G)], srcr.at[gn],
                            sem_i).wait()
                        pltpu.async_copy(spm.at[srcr.at[gn].at[0]],
                                         rows.at[1 - b], sem_a.at[1 - b])

                    @pl.when(g + 2 < ng)
                    def _pref():
                        off = pl.multiple_of((g + 2) * G, G)
                        pltpu.async_copy(
                            src_idx.at[w].at[pl.ds(off, G)],
                            srcr.at[gp], sem_i)
                pltpu.make_async_copy(table.at[pl.ds(0, CHUNK)],
                                      rows.at[b], sem_a.at[b]).wait()
                pltpu.async_copy(
                    rows.at[b],
                    msg.at[pl.ds((base + k) * CHUNK, CHUNK)],
                    sem_b.at[b])
            return carry

        lax.fori_loop(0, ng, grp, 0)
        pltpu.make_async_copy(table.at[pl.ds(0, CHUNK)],
                              rows.at[(cpw - 1) % 2],
                              sem_b.at[(cpw - 1) % 2]).wait()
        plsc.subcore_barrier()
        pltpu.sync_copy(zeros, spm.at[pl.ds(s * ZROWS, ZROWS)])
        plsc.subcore_barrier()

        pltpu.async_copy(msg.at[pl.ds(base * CHUNK, CHUNK)], rows.at[0],
                         sem_a.at[0])

        def pair(t, carry):
            for db in range(2):
                k = t * 2 + db
                b = db
                @pl.when(k >= 1)
                def _drain():
                    pltpu.make_async_copy(msg.at[pl.ds(0, CHUNK)],
                                          rows.at[1 - b],
                                          sem_b.at[1 - b]).wait()

                @pl.when(k + 1 < cpw)
                def _next():
                    pltpu.async_copy(
                        msg.at[pl.ds((base + k + 1) * CHUNK, CHUNK)],
                        rows.at[1 - b], sem_a.at[1 - b])
                pltpu.make_async_copy(msg.at[pl.ds(0, CHUNK)],
                                      rows.at[b], sem_a.at[b]).wait()
                pltpu.async_copy(rows.at[b], spm.at[dst_v.at[k]],
                                 sem_b.at[b], add=True)
            return carry

        lax.fori_loop(0, cpw // 2, pair, 0)
        pltpu.make_async_copy(msg.at[pl.ds(0, CHUNK)],
                              rows.at[(cpw - 1) % 2],
                              sem_b.at[(cpw - 1) % 2]).wait()
        plsc.subcore_barrier()
        pltpu.sync_copy(spm.at[pl.ds(s * ZROWS, ZROWS)],
                        out.at[c].at[pl.ds(s * ZROWS, ZROWS)])

    return layer_k


_BLK = 2000


def _mid_body(p_ref, w1_ref, b1_ref, w2_ref, o_ref):
    a = p_ref[0] + p_ref[1]
    h = jnp.maximum(
        jax.lax.dot(a, w1_ref[...], precision=_HI) + b1_ref[...], 0.0)
    o_ref[...] = jax.lax.dot(h, w2_ref[...], precision=_HI)


def _dense_mid(p, W1, b1, W2):
    n = N_NODES
    return pl.pallas_call(
        _mid_body,
        grid=(n // _BLK,),
        in_specs=[
            pl.BlockSpec((NC, _BLK, D), lambda i: (0, i, 0)),
            pl.BlockSpec((D, D), lambda i: (0, 0)),
            pl.BlockSpec((1, D), lambda i: (0, 0)),
            pl.BlockSpec((D, D), lambda i: (0, 0)),
        ],
        out_specs=pl.BlockSpec((_BLK, D), lambda i: (i, 0)),
        out_shape=jax.ShapeDtypeStruct((n, D), jnp.float32),
    )(p, W1, b1.reshape(1, D), W2)


def _fin_body(n, p_ref, b2_ref, wp1_ref, bp1_ref, wp2_ref, bp2_ref, o_ref,
              acc_ref):
    i = pl.program_id(0)

    @pl.when(i == 0)
    def _zero():
        acc_ref[...] = jnp.zeros_like(acc_ref)

    h2 = jnp.maximum(p_ref[0] + p_ref[1] + b2_ref[...], 0.0)
    acc_ref[...] += jnp.sum(h2, axis=0, keepdims=True)

    @pl.when(i == pl.num_programs(0) - 1)
    def _head():
        g = acc_ref[...] * (1.0 / n)
        hp = jnp.maximum(
            jax.lax.dot(g, wp1_ref[...], precision=_HI) + bp1_ref[...], 0.0)
        o_ref[...] = jax.lax.dot(hp, wp2_ref[...], precision=_HI) \
            + bp2_ref[...]


def _dense_final(p, b2, Wp1, bp1, Wp2, bp2):
    n = N_NODES
    return pl.pallas_call(
        functools.partial(_fin_body, n),
        grid=(n // _BLK,),
        in_specs=[
            pl.BlockSpec((NC, _BLK, D), lambda i: (0, i, 0)),
            pl.BlockSpec((1, D), lambda i: (0, 0)),
            pl.BlockSpec((D, D), lambda i: (0, 0)),
            pl.BlockSpec((1, D), lambda i: (0, 0)),
            pl.BlockSpec((D, 16), lambda i: (0, 0)),
            pl.BlockSpec((1, 16), lambda i: (0, 0)),
        ],
        out_specs=pl.BlockSpec((1, 16), lambda i: (0, 0)),
        out_shape=jax.ShapeDtypeStruct((1, 16), jnp.float32),
        scratch_shapes=[pltpu.VMEM((1, D), jnp.float32)],
    )(p, b2.reshape(1, D), Wp1, bp1.reshape(1, D), Wp2, bp2.reshape(1, 16))


def kernel(x, edge_index, W1, b1, W2, b2, Wp1, bp1, Wp2, bp2):
    src = edge_index[0]
    dst = edge_index[1]
    e = src.shape[0]
    cpw = -(-e // (NW * CHUNK))
    cpw = -(-cpw // G) * G
    e_pad = NW * cpw * CHUNK
    pad = e_pad - e
    src_p = jnp.concatenate(
        [src, jnp.zeros((pad,), jnp.int32)]).reshape(NW, cpw, CHUNK)
    dst_p = jnp.concatenate(
        [dst, jnp.full((pad,), N_NODES, jnp.int32)]).reshape(NW, cpw, CHUNK)
    zeros = jnp.zeros((ZROWS, D), jnp.float32)

    layer = _make_layer(cpw)
    p1, _ = layer(x, src_p, dst_p, zeros)
    hw2 = _dense_mid(p1, W1, b1, W2)
    p2, _ = layer(hw2, src_p, dst_p, zeros)
    out = _dense_final(p2, b2, Wp1, bp1, Wp2, bp2)
    return out.reshape(16)


# speedup vs baseline: 1.1135x; 1.0372x over previous
import functools

import jax
import jax.numpy as jnp
from jax import lax
from jax.experimental import pallas as pl
from jax.experimental.pallas import tpu as pltpu
from jax.experimental.pallas import tpu_sc as plsc

N_NODES = 10000
D = 128
NC = 2
NS = 16
NW = NC * NS
CHUNK = 128
G = 8
ZROWS = 632
N_ACC = NS * ZROWS

_HI = jax.lax.Precision.HIGHEST


def _make_layer(cpw):
    mesh = plsc.VectorSubcoreMesh(core_axis_name="c", subcore_axis_name="s")
    ng = cpw // G
    assert cpw % G == 0

    @functools.partial(
        pl.kernel,
        out_type=(jax.ShapeDtypeStruct((NC, N_ACC, D), jnp.float32),
                  jax.ShapeDtypeStruct((NW * cpw * CHUNK, D), jnp.float32)),
        mesh=mesh,
        scratch_types=[
            pltpu.VMEM((2, G, CHUNK), jnp.int32),
            pltpu.VMEM((cpw, CHUNK), jnp.int32),
            pltpu.VMEM((2, CHUNK, D), jnp.float32),
            pltpu.VMEM_SHARED((N_ACC, D), jnp.float32),
            pltpu.SemaphoreType.DMA((2,)),
            pltpu.SemaphoreType.DMA((2,)),
            pltpu.SemaphoreType.DMA,
        ],
    )
    def layer_k(table, src_idx, dst_idx, out, msg, srcr, dst_v,
                rows, spm, sem_a, sem_b, sem_i):
        c = lax.axis_index("c")
        s = lax.axis_index("s")
        w = c * NS + s
        base = w * cpw
        pltpu.sync_copy(src_idx.at[w].at[pl.ds(0, G)], srcr.at[0])
        pltpu.async_copy(src_idx.at[w].at[pl.ds(G, G)], srcr.at[1], sem_i)
        pltpu.sync_copy(dst_idx.at[w], dst_v)
        @pl.when(s < 15)
        def _stage():
            pltpu.sync_copy(table.at[pl.ds(s * 640, 640)],
                            spm.at[pl.ds(s * 640, 640)])

        @pl.when(s == 15)
        def _stage_tail():
            pltpu.sync_copy(table.at[pl.ds(9600, 400)],
                            spm.at[pl.ds(9600, 400)])
        plsc.subcore_barrier()

        pltpu.async_copy(spm.at[srcr.at[0].at[0]], rows.at[0], sem_a.at[0])

        def grp(g, carry):
            gp = g % 2
            gn = (g + 1) % 2
            for r in range(G):
                k = g * G + r
                b = r % 2
                @pl.when(k >= 1)
                def _drain():
                    pltpu.make_async_copy(table.at[pl.ds(0, CHUNK)],
                                          rows.at[1 - b],
                                          sem_b.at[1 - b]).wait()

                if r < G - 1:
                    pltpu.async_copy(spm.at[srcr.at[gp].at[r + 1]],
                                     rows.at[1 - b], sem_a.at[1 - b])
                else:
                    @pl.when(g + 1 < ng)
                    def _cross():
                        pltpu.make_async_copy(
                            src_idx.at[0].at[pl.ds(0, G)], srcr.at[gn],
                            sem_i).wait()
                        pltpu.async_copy(spm.at[srcr.at[gn].at[0]],
                                         rows.at[1 - b], sem_a.at[1 - b])

                    @pl.when(g + 2 < ng)
                    def _pref():
                        off = pl.multiple_of((g + 2) * G, G)
                        pltpu.async_copy(
                            src_idx.at[w].at[pl.ds(off, G)],
                            srcr.at[gp], sem_i)
                pltpu.make_async_copy(table.at[pl.ds(0, CHUNK)],
                                      rows.at[b], sem_a.at[b]).wait()
                pltpu.async_copy(
                    rows.at[b],
                    msg.at[pl.ds((base + k) * CHUNK, CHUNK)],
                    sem_b.at[b])
            return carry

        lax.fori_loop(0, ng, grp, 0)
        pltpu.make_async_copy(table.at[pl.ds(0, CHUNK)],
                              rows.at[(cpw - 1) % 2],
                              sem_b.at[(cpw - 1) % 2]).wait()
        plsc.subcore_barrier()

        pltpu.async_copy(msg.at[pl.ds(base * CHUNK, CHUNK)], rows.at[0],
                         sem_a.at[0])

        def pair(t, carry):
            for db in range(2):
                k = t * 2 + db
                b = db
                @pl.when(k >= 1)
                def _drain():
                    pltpu.make_async_copy(msg.at[pl.ds(0, CHUNK)],
                                          rows.at[1 - b],
                                          sem_b.at[1 - b]).wait()

                @pl.when(k + 1 < cpw)
                def _next():
                    pltpu.async_copy(
                        msg.at[pl.ds((base + k + 1) * CHUNK, CHUNK)],
                        rows.at[1 - b], sem_a.at[1 - b])
                pltpu.make_async_copy(msg.at[pl.ds(0, CHUNK)],
                                      rows.at[b], sem_a.at[b]).wait()
                pltpu.async_copy(rows.at[b], spm.at[dst_v.at[k]],
                                 sem_b.at[b], add=True)
            return carry

        lax.fori_loop(0, cpw // 2, pair, 0)
        pltpu.make_async_copy(msg.at[pl.ds(0, CHUNK)],
                              rows.at[(cpw - 1) % 2],
                              sem_b.at[(cpw - 1) % 2]).wait()
        plsc.subcore_barrier()
        pltpu.sync_copy(spm.at[pl.ds(s * ZROWS, ZROWS)],
                        out.at[c].at[pl.ds(s * ZROWS, ZROWS)])

    return layer_k


_BLK = 2000


def _mid_body(p_ref, t_ref, w1_ref, b1_ref, w2_ref, o_ref):
    a = p_ref[0] + p_ref[1] - 2.0 * t_ref[...]
    h = jnp.maximum(
        jax.lax.dot(a, w1_ref[...], precision=_HI) + b1_ref[...], 0.0)
    o_ref[...] = jax.lax.dot(h, w2_ref[...], precision=_HI)


def _dense_mid(p, t, W1, b1, W2):
    n = N_NODES
    return pl.pallas_call(
        _mid_body,
        grid=(n // _BLK,),
        in_specs=[
            pl.BlockSpec((NC, _BLK, D), lambda i: (0, i, 0)),
            pl.BlockSpec((_BLK, D), lambda i: (i, 0)),
            pl.BlockSpec((D, D), lambda i: (0, 0)),
            pl.BlockSpec((1, D), lambda i: (0, 0)),
            pl.BlockSpec((D, D), lambda i: (0, 0)),
        ],
        out_specs=pl.BlockSpec((_BLK, D), lambda i: (i, 0)),
        out_shape=jax.ShapeDtypeStruct((n, D), jnp.float32),
    )(p, t, W1, b1.reshape(1, D), W2)


def _fin_body(n, p_ref, t_ref, b2_ref, wp1_ref, bp1_ref, wp2_ref, bp2_ref,
              o_ref, acc_ref):
    i = pl.program_id(0)

    @pl.when(i == 0)
    def _zero():
        acc_ref[...] = jnp.zeros_like(acc_ref)

    h2 = jnp.maximum(
        p_ref[0] + p_ref[1] - 2.0 * t_ref[...] + b2_ref[...], 0.0)
    acc_ref[...] += jnp.sum(h2, axis=0, keepdims=True)

    @pl.when(i == pl.num_programs(0) - 1)
    def _head():
        g = acc_ref[...] * (1.0 / n)
        hp = jnp.maximum(
            jax.lax.dot(g, wp1_ref[...], precision=_HI) + bp1_ref[...], 0.0)
        o_ref[...] = jax.lax.dot(hp, wp2_ref[...], precision=_HI) \
            + bp2_ref[...]


def _dense_final(p, t, b2, Wp1, bp1, Wp2, bp2):
    n = N_NODES
    return pl.pallas_call(
        functools.partial(_fin_body, n),
        grid=(n // _BLK,),
        in_specs=[
            pl.BlockSpec((NC, _BLK, D), lambda i: (0, i, 0)),
            pl.BlockSpec((_BLK, D), lambda i: (i, 0)),
            pl.BlockSpec((1, D), lambda i: (0, 0)),
            pl.BlockSpec((D, D), lambda i: (0, 0)),
            pl.BlockSpec((1, D), lambda i: (0, 0)),
            pl.BlockSpec((D, 16), lambda i: (0, 0)),
            pl.BlockSpec((1, 16), lambda i: (0, 0)),
        ],
        out_specs=pl.BlockSpec((1, 16), lambda i: (0, 0)),
        out_shape=jax.ShapeDtypeStruct((1, 16), jnp.float32),
        scratch_shapes=[pltpu.VMEM((1, D), jnp.float32)],
    )(p, t, b2.reshape(1, D), Wp1, bp1.reshape(1, D), Wp2,
      bp2.reshape(1, 16))


def kernel(x, edge_index, W1, b1, W2, b2, Wp1, bp1, Wp2, bp2):
    src = edge_index[0]
    dst = edge_index[1]
    e = src.shape[0]
    cpw = -(-e // (NW * CHUNK))
    cpw = -(-cpw // G) * G
    e_pad = NW * cpw * CHUNK
    pad = e_pad - e
    src_p = jnp.concatenate(
        [src, jnp.zeros((pad,), jnp.int32)]).reshape(NW, cpw, CHUNK)
    dst_p = jnp.concatenate(
        [dst, jnp.full((pad,), N_NODES, jnp.int32)]).reshape(NW, cpw, CHUNK)
    layer = _make_layer(cpw)
    p1, _ = layer(x, src_p, dst_p)
    hw2 = _dense_mid(p1, x, W1, b1, W2)
    p2, _ = layer(hw2, src_p, dst_p)
    out = _dense_final(p2, hw2, b2, Wp1, bp1, Wp2, bp2)
    return out.reshape(16)


# speedup vs baseline: 1.1261x; 1.0113x over previous
import functools

import jax
import jax.numpy as jnp
from jax import lax
from jax.experimental import pallas as pl
from jax.experimental.pallas import tpu as pltpu
from jax.experimental.pallas import tpu_sc as plsc

N_NODES = 10000
D = 128
NC = 2
NS = 16
NW = NC * NS
CHUNK = 128
G = 8
ZROWS = 632
N_ACC = NS * ZROWS

_HI = jax.lax.Precision.HIGHEST


def _make_layer(cpw):
    mesh = plsc.VectorSubcoreMesh(core_axis_name="c", subcore_axis_name="s")
    ng = cpw // G
    assert cpw % G == 0

    @functools.partial(
        pl.kernel,
        out_type=(jax.ShapeDtypeStruct((NC, N_ACC, D), jnp.float32),
                  jax.ShapeDtypeStruct((NW * cpw * CHUNK, D), jnp.float32)),
        mesh=mesh,
        scratch_types=[
            pltpu.VMEM((2, G, CHUNK), jnp.int32),
            pltpu.VMEM((cpw, CHUNK), jnp.int32),
            pltpu.VMEM((2, CHUNK, D), jnp.float32),
            pltpu.VMEM_SHARED((N_ACC, D), jnp.float32),
            pltpu.SemaphoreType.DMA((2,)),
            pltpu.SemaphoreType.DMA((2,)),
            pltpu.SemaphoreType.DMA,
            pltpu.SemaphoreType.DMA,
        ],
    )
    def layer_k(table, src_idx, dst_idx, out, msg, srcr, dst_v,
                rows, spm, sem_a, sem_b, sem_i, sem_d):
        c = lax.axis_index("c")
        s = lax.axis_index("s")
        w = c * NS + s
        base = w * cpw
        pltpu.sync_copy(src_idx.at[w].at[pl.ds(0, G)], srcr.at[0])
        pltpu.async_copy(src_idx.at[w].at[pl.ds(G, G)], srcr.at[1], sem_i)
        pltpu.async_copy(dst_idx.at[w], dst_v, sem_d)
        @pl.when(s < 15)
        def _stage():
            pltpu.sync_copy(table.at[pl.ds(s * 640, 640)],
                            spm.at[pl.ds(s * 640, 640)])

        @pl.when(s == 15)
        def _stage_tail():
            pltpu.sync_copy(table.at[pl.ds(9600, 400)],
                            spm.at[pl.ds(9600, 400)])
        plsc.subcore_barrier()

        pltpu.async_copy(spm.at[srcr.at[0].at[0]], rows.at[0], sem_a.at[0])

        def grp(g, carry):
            gp = g % 2
            gn = (g + 1) % 2
            for r in range(G):
                k = g * G + r
                b = r % 2
                @pl.when(k >= 1)
                def _drain():
                    pltpu.make_async_copy(table.at[pl.ds(0, CHUNK)],
                                          rows.at[1 - b],
                                          sem_b.at[1 - b]).wait()

                if r < G - 1:
                    pltpu.async_copy(spm.at[srcr.at[gp].at[r + 1]],
                                     rows.at[1 - b], sem_a.at[1 - b])
                else:
                    @pl.when(g + 1 < ng)
                    def _cross():
                        pltpu.make_async_copy(
                            src_idx.at[0].at[pl.ds(0, G)], srcr.at[gn],
                            sem_i).wait()
                        pltpu.async_copy(spm.at[srcr.at[gn].at[0]],
                                         rows.at[1 - b], sem_a.at[1 - b])

                    @pl.when(g + 2 < ng)
                    def _pref():
                        off = pl.multiple_of((g + 2) * G, G)
                        pltpu.async_copy(
                            src_idx.at[w].at[pl.ds(off, G)],
                            srcr.at[gp], sem_i)
                pltpu.make_async_copy(table.at[pl.ds(0, CHUNK)],
                                      rows.at[b], sem_a.at[b]).wait()
                pltpu.async_copy(
                    rows.at[b],
                    msg.at[pl.ds((base + k) * CHUNK, CHUNK)],
                    sem_b.at[b])
            return carry

        lax.fori_loop(0, ng, grp, 0)
        pltpu.make_async_copy(table.at[pl.ds(0, CHUNK)],
                              rows.at[(cpw - 1) % 2],
                              sem_b.at[(cpw - 1) % 2]).wait()
        pltpu.async_copy(msg.at[pl.ds(base * CHUNK, CHUNK)], rows.at[0],
                         sem_a.at[0])
        plsc.subcore_barrier()
        pltpu.make_async_copy(dst_idx.at[0], dst_v, sem_d).wait()


        def pair(t, carry):
            for db in range(2):
                k = t * 2 + db
                b = db
                @pl.when(k >= 1)
                def _drain():
                    pltpu.make_async_copy(msg.at[pl.ds(0, CHUNK)],
                                          rows.at[1 - b],
                                          sem_b.at[1 - b]).wait()

                @pl.when(k + 1 < cpw)
                def _next():
                    pltpu.async_copy(
                        msg.at[pl.ds((base + k + 1) * CHUNK, CHUNK)],
                        rows.at[1 - b], sem_a.at[1 - b])
                pltpu.make_async_copy(msg.at[pl.ds(0, CHUNK)],
                                      rows.at[b], sem_a.at[b]).wait()
                pltpu.async_copy(rows.at[b], spm.at[dst_v.at[k]],
                                 sem_b.at[b], add=True)
            return carry

        lax.fori_loop(0, cpw // 2, pair, 0)
        pltpu.make_async_copy(msg.at[pl.ds(0, CHUNK)],
                              rows.at[(cpw - 1) % 2],
                              sem_b.at[(cpw - 1) % 2]).wait()
        plsc.subcore_barrier()
        pltpu.sync_copy(spm.at[pl.ds(s * ZROWS, ZROWS)],
                        out.at[c].at[pl.ds(s * ZROWS, ZROWS)])

    return layer_k


_BLK = 2000


def _mid_body(p_ref, t_ref, w1_ref, b1_ref, w2_ref, o_ref):
    a = p_ref[0] + p_ref[1] - 2.0 * t_ref[...]
    h = jnp.maximum(
        jax.lax.dot(a, w1_ref[...], precision=_HI) + b1_ref[...], 0.0)
    o_ref[...] = jax.lax.dot(h, w2_ref[...], precision=_HI)


def _dense_mid(p, t, W1, b1, W2):
    n = N_NODES
    return pl.pallas_call(
        _mid_body,
        grid=(n // _BLK,),
        in_specs=[
            pl.BlockSpec((NC, _BLK, D), lambda i: (0, i, 0)),
            pl.BlockSpec((_BLK, D), lambda i: (i, 0)),
            pl.BlockSpec((D, D), lambda i: (0, 0)),
            pl.BlockSpec((1, D), lambda i: (0, 0)),
            pl.BlockSpec((D, D), lambda i: (0, 0)),
        ],
        out_specs=pl.BlockSpec((_BLK, D), lambda i: (i, 0)),
        out_shape=jax.ShapeDtypeStruct((n, D), jnp.float32),
    )(p, t, W1, b1.reshape(1, D), W2)


def _fin_body(n, p_ref, t_ref, b2_ref, wp1_ref, bp1_ref, wp2_ref, bp2_ref,
              o_ref, acc_ref):
    i = pl.program_id(0)

    @pl.when(i == 0)
    def _zero():
        acc_ref[...] = jnp.zeros_like(acc_ref)

    h2 = jnp.maximum(
        p_ref[0] + p_ref[1] - 2.0 * t_ref[...] + b2_ref[...], 0.0)
    acc_ref[...] += jnp.sum(h2, axis=0, keepdims=True)

    @pl.when(i == pl.num_programs(0) - 1)
    def _head():
        g = acc_ref[...] * (1.0 / n)
        hp = jnp.maximum(
            jax.lax.dot(g, wp1_ref[...], precision=_HI) + bp1_ref[...], 0.0)
        o_ref[...] = jax.lax.dot(hp, wp2_ref[...], precision=_HI) \
            + bp2_ref[...]


def _dense_final(p, t, b2, Wp1, bp1, Wp2, bp2):
    n = N_NODES
    return pl.pallas_call(
        functools.partial(_fin_body, n),
        grid=(n // _BLK,),
        in_specs=[
            pl.BlockSpec((NC, _BLK, D), lambda i: (0, i, 0)),
            pl.BlockSpec((_BLK, D), lambda i: (i, 0)),
            pl.BlockSpec((1, D), lambda i: (0, 0)),
            pl.BlockSpec((D, D), lambda i: (0, 0)),
            pl.BlockSpec((1, D), lambda i: (0, 0)),
            pl.BlockSpec((D, 16), lambda i: (0, 0)),
            pl.BlockSpec((1, 16), lambda i: (0, 0)),
        ],
        out_specs=pl.BlockSpec((1, 16), lambda i: (0, 0)),
        out_shape=jax.ShapeDtypeStruct((1, 16), jnp.float32),
        scratch_shapes=[pltpu.VMEM((1, D), jnp.float32)],
    )(p, t, b2.reshape(1, D), Wp1, bp1.reshape(1, D), Wp2,
      bp2.reshape(1, 16))


def kernel(x, edge_index, W1, b1, W2, b2, Wp1, bp1, Wp2, bp2):
    src = edge_index[0]
    dst = edge_index[1]
    e = src.shape[0]
    cpw = -(-e // (NW * CHUNK))
    cpw = -(-cpw // G) * G
    e_pad = NW * cpw * CHUNK
    pad = e_pad - e
    src_p = jnp.concatenate(
        [src, jnp.zeros((pad,), jnp.int32)]).reshape(NW, cpw, CHUNK)
    dst_p = jnp.concatenate(
        [dst, jnp.full((pad,), N_NODES, jnp.int32)]).reshape(NW, cpw, CHUNK)
    layer = _make_layer(cpw)
    p1, _ = layer(x, src_p, dst_p)
    hw2 = _dense_mid(p1, x, W1, b1, W2)
    p2, _ = layer(hw2, src_p, dst_p)
    out = _dense_final(p2, hw2, b2, Wp1, bp1, Wp2, bp2)
    return out.reshape(16)


# speedup vs baseline: 1.1534x; 1.0243x over previous
import functools

import jax
import jax.numpy as jnp
from jax import lax
from jax.experimental import pallas as pl
from jax.experimental.pallas import tpu as pltpu
from jax.experimental.pallas import tpu_sc as plsc

N_NODES = 10000
D = 128
NC = 2
NS = 16
NW = NC * NS
CHUNK = 128
G = 8
ZROWS = 632
N_ACC = NS * ZROWS

def _make_layer(cpw):
    mesh = plsc.VectorSubcoreMesh(core_axis_name="c", subcore_axis_name="s")
    ng = cpw // G
    assert cpw % G == 0

    @functools.partial(
        pl.kernel,
        out_type=(jax.ShapeDtypeStruct((NC, N_ACC, D), jnp.float32),
                  jax.ShapeDtypeStruct((NW * cpw * CHUNK, D), jnp.float32)),
        mesh=mesh,
        scratch_types=[
            pltpu.VMEM((2, G, CHUNK), jnp.int32),
            pltpu.VMEM((cpw, CHUNK), jnp.int32),
            pltpu.VMEM((2, CHUNK, D), jnp.float32),
            pltpu.VMEM_SHARED((N_ACC, D), jnp.float32),
            pltpu.SemaphoreType.DMA((2,)),
            pltpu.SemaphoreType.DMA((2,)),
            pltpu.SemaphoreType.DMA,
            pltpu.SemaphoreType.DMA,
        ],
    )
    def layer_k(table, src_idx, dst_idx, out, msg, srcr, dst_v,
                rows, spm, sem_a, sem_b, sem_i, sem_d):
        c = lax.axis_index("c")
        s = lax.axis_index("s")
        w = c * NS + s
        base = w * cpw
        pltpu.sync_copy(src_idx.at[w].at[pl.ds(0, G)], srcr.at[0])
        pltpu.async_copy(src_idx.at[w].at[pl.ds(G, G)], srcr.at[1], sem_i)
        pltpu.async_copy(dst_idx.at[w], dst_v, sem_d)
        @pl.when(s < 15)
        def _stage():
            pltpu.sync_copy(table.at[pl.ds(s * 640, 640)],
                            spm.at[pl.ds(s * 640, 640)])

        @pl.when(s == 15)
        def _stage_tail():
            pltpu.sync_copy(table.at[pl.ds(9600, 400)],
                            spm.at[pl.ds(9600, 400)])
        plsc.subcore_barrier()

        pltpu.async_copy(spm.at[srcr.at[0].at[0]], rows.at[0], sem_a.at[0])

        def grp(g, carry):
            gp = g % 2
            gn = (g + 1) % 2
            for r in range(G):
                k = g * G + r
                b = r % 2
                @pl.when(k >= 1)
                def _drain():
                    pltpu.make_async_copy(table.at[pl.ds(0, CHUNK)],
                                          rows.at[1 - b],
                                          sem_b.at[1 - b]).wait()

                if r < G - 1:
                    pltpu.async_copy(spm.at[srcr.at[gp].at[r + 1]],
                                     rows.at[1 - b], sem_a.at[1 - b])
                else:
                    @pl.when(g + 1 < ng)
                    def _cross():
                        pltpu.make_async_copy(
                            src_idx.at[0].at[pl.ds(0, G)], srcr.at[gn],
                            sem_i).wait()
                        pltpu.async_copy(spm.at[srcr.at[gn].at[0]],
                                         rows.at[1 - b], sem_a.at[1 - b])

                    @pl.when(g + 2 < ng)
                    def _pref():
                        off = pl.multiple_of((g + 2) * G, G)
                        pltpu.async_copy(
                            src_idx.at[w].at[pl.ds(off, G)],
                            srcr.at[gp], sem_i)
                pltpu.make_async_copy(table.at[pl.ds(0, CHUNK)],
                                      rows.at[b], sem_a.at[b]).wait()
                pltpu.async_copy(
                    rows.at[b],
                    msg.at[pl.ds((base + k) * CHUNK, CHUNK)],
                    sem_b.at[b])
            return carry

        lax.fori_loop(0, ng, grp, 0)
        pltpu.make_async_copy(table.at[pl.ds(0, CHUNK)],
                              rows.at[(cpw - 1) % 2],
                              sem_b.at[(cpw - 1) % 2]).wait()
        pltpu.async_copy(msg.at[pl.ds(base * CHUNK, CHUNK)], rows.at[0],
                         sem_a.at[0])
        plsc.subcore_barrier()
        pltpu.make_async_copy(dst_idx.at[0], dst_v, sem_d).wait()


        def pair(t, carry):
            for db in range(2):
                k = t * 2 + db
                b = db
                @pl.when(k >= 1)
                def _drain():
                    pltpu.make_async_copy(msg.at[pl.ds(0, CHUNK)],
                                          rows.at[1 - b],
                                          sem_b.at[1 - b]).wait()

                @pl.when(k + 1 < cpw)
                def _next():
                    pltpu.async_copy(
                        msg.at[pl.ds((base + k + 1) * CHUNK, CHUNK)],
                        rows.at[1 - b], sem_a.at[1 - b])
                pltpu.make_async_copy(msg.at[pl.ds(0, CHUNK)],
                                      rows.at[b], sem_a.at[b]).wait()
                pltpu.async_copy(rows.at[b], spm.at[dst_v.at[k]],
                                 sem_b.at[b], add=True)
            return carry

        lax.fori_loop(0, cpw // 2, pair, 0)
        pltpu.make_async_copy(msg.at[pl.ds(0, CHUNK)],
                              rows.at[(cpw - 1) % 2],
                              sem_b.at[(cpw - 1) % 2]).wait()
        plsc.subcore_barrier()
        pltpu.sync_copy(spm.at[pl.ds(s * ZROWS, ZROWS)],
                        out.at[c].at[pl.ds(s * ZROWS, ZROWS)])

    return layer_k


_BLK = 2000


def _mid_body(p_ref, t_ref, w1_ref, b1_ref, o_ref):
    a = p_ref[0] + p_ref[1] - 2.0 * t_ref[...]
    o_ref[...] = jnp.maximum(jax.lax.dot(a, w1_ref[...]) + b1_ref[...], 0.0)


def _dense_mid(p, t, W1, b1):
    n = N_NODES
    return pl.pallas_call(
        _mid_body,
        grid=(n // _BLK,),
        in_specs=[
            pl.BlockSpec((NC, _BLK, D), lambda i: (0, i, 0)),
            pl.BlockSpec((_BLK, D), lambda i: (i, 0)),
            pl.BlockSpec((D, D), lambda i: (0, 0)),
            pl.BlockSpec((1, D), lambda i: (0, 0)),
        ],
        out_specs=pl.BlockSpec((_BLK, D), lambda i: (i, 0)),
        out_shape=jax.ShapeDtypeStruct((n, D), jnp.float32),
    )(p, t, W1, b1.reshape(1, D))


def _fin_body(n, p_ref, t_ref, w2_ref, b2_ref, wp1_ref, bp1_ref, wp2_ref,
              bp2_ref, o_ref, acc_ref):
    i = pl.program_id(0)

    @pl.when(i == 0)
    def _zero():
        acc_ref[...] = jnp.zeros_like(acc_ref)

    agg2 = p_ref[0] + p_ref[1] - 2.0 * t_ref[...]
    h2 = jnp.maximum(jax.lax.dot(agg2, w2_ref[...]) + b2_ref[...], 0.0)
    acc_ref[...] += jnp.sum(h2, axis=0, keepdims=True)

    @pl.when(i == pl.num_programs(0) - 1)
    def _head():
        g = acc_ref[...] * (1.0 / n)
        hp = jnp.maximum(
            jax.lax.dot(g, wp1_ref[...]) + bp1_ref[...], 0.0)
        o_ref[...] = jax.lax.dot(hp, wp2_ref[...]) + bp2_ref[...]


def _dense_final(p, t, W2, b2, Wp1, bp1, Wp2, bp2):
    n = N_NODES
    return pl.pallas_call(
        functools.partial(_fin_body, n),
        grid=(n // _BLK,),
        in_specs=[
            pl.BlockSpec((NC, _BLK, D), lambda i: (0, i, 0)),
            pl.BlockSpec((_BLK, D), lambda i: (i, 0)),
            pl.BlockSpec((D, D), lambda i: (0, 0)),
            pl.BlockSpec((1, D), lambda i: (0, 0)),
            pl.BlockSpec((D, D), lambda i: (0, 0)),
            pl.BlockSpec((1, D), lambda i: (0, 0)),
            pl.BlockSpec((D, 16), lambda i: (0, 0)),
            pl.BlockSpec((1, 16), lambda i: (0, 0)),
        ],
        out_specs=pl.BlockSpec((1, 16), lambda i: (0, 0)),
        out_shape=jax.ShapeDtypeStruct((1, 16), jnp.float32),
        scratch_shapes=[pltpu.VMEM((1, D), jnp.float32)],
    )(p, t, W2, b2.reshape(1, D), Wp1, bp1.reshape(1, D), Wp2,
      bp2.reshape(1, 16))


def kernel(x, edge_index, W1, b1, W2, b2, Wp1, bp1, Wp2, bp2):
    src = edge_index[0]
    dst = edge_index[1]
    e = src.shape[0]
    cpw = -(-e // (NW * CHUNK))
    cpw = -(-cpw // G) * G
    e_pad = NW * cpw * CHUNK
    pad = e_pad - e
    src_p = jnp.concatenate(
        [src, jnp.zeros((pad,), jnp.int32)]).reshape(NW, cpw, CHUNK)
    dst_p = jnp.concatenate(
        [dst, jnp.full((pad,), N_NODES, jnp.int32)]).reshape(NW, cpw, CHUNK)
    layer = _make_layer(cpw)
    p1, _ = layer(x, src_p, dst_p)
    h = _dense_mid(p1, x, W1, b1)
    p2, _ = layer(h, src_p, dst_p)
    out = _dense_final(p2, h, W2, b2, Wp1, bp1, Wp2, bp2)
    return out.reshape(16)
